# baseline (speedup 1.0000x reference)
import functools

import jax
import jax.numpy as jnp
from jax.experimental import pallas as pl
from jax.experimental.pallas import tpu as pltpu


def _moe_body(x_ref, gw_ref, gb_ref, w_ref, b_ref, out_ref, g_scr, *, n_experts):
    xb = x_ref[...].astype(jnp.bfloat16)
    m_idx = pl.program_id(1)

    @pl.when(pl.program_id(0) == 0)
    def _gate():
        logits = jax.lax.dot_general(
            xb,
            gw_ref[...].astype(jnp.bfloat16),
            (((1,), (1,)), ((), ())),
            preferred_element_type=jnp.float32,
        )
        logits = logits + gb_ref[...]
        mx = jnp.max(logits, axis=-1, keepdims=True)
        p = jnp.exp(logits - mx)
        g_scr[m_idx] = (p / jnp.sum(p, axis=-1, keepdims=True)).astype(jnp.bfloat16)

    g = g_scr[m_idx].astype(jnp.float32)
    acc = jnp.dot(g, b_ref[...], preferred_element_type=jnp.float32)
    for e in range(n_experts):
        ye = jnp.dot(xb, w_ref[e].astype(jnp.bfloat16), preferred_element_type=jnp.float32)
        acc = acc + g[:, e : e + 1] * ye
    out_ref[...] = acc


def kernel(x, gate_W, gate_b, expert_W, expert_b):
    n_tok, k = x.shape
    n_exp, _, n_out = expert_W.shape

    bm = min(1024, n_tok)
    bn = min(256, n_out)
    grid = (n_out // bn, n_tok // bm)

    gb2 = gate_b.reshape(1, n_exp)
    gw_t = gate_W.T

    body = functools.partial(_moe_body, n_experts=n_exp)
    return pl.pallas_call(
        body,
        grid=grid,
        in_specs=[
            pl.BlockSpec((bm, k), lambda n, m: (m, 0)),
            pl.BlockSpec((n_exp, k), lambda n, m: (0, 0)),
            pl.BlockSpec((1, n_exp), lambda n, m: (0, 0)),
            pl.BlockSpec((n_exp, k, bn), lambda n, m: (0, 0, n)),
            pl.BlockSpec((n_exp, bn), lambda n, m: (0, n)),
        ],
        out_specs=pl.BlockSpec((bm, bn), lambda n, m: (m, n)),
        out_shape=jax.ShapeDtypeStruct((n_tok, n_out), jnp.float32),
        scratch_shapes=[pltpu.VMEM((n_tok // bm, bm, n_exp), jnp.bfloat16)],
        compiler_params=pltpu.CompilerParams(
            dimension_semantics=("arbitrary", "arbitrary"),
        ),
    )(x, gw_t, gb2, expert_W, expert_b)


# speedup vs baseline: 1.0141x; 1.0141x over previous
import functools

import jax
import jax.numpy as jnp
from jax.experimental import pallas as pl
from jax.experimental.pallas import tpu as pltpu


def _gate_body(x_ref, gw_ref, gb_ref, g_ref):
    logits = jnp.dot(
        x_ref[...].astype(jnp.bfloat16),
        gw_ref[...].astype(jnp.bfloat16),
        preferred_element_type=jnp.float32,
    )
    logits = logits + gb_ref[...]
    m = jnp.max(logits, axis=-1, keepdims=True)
    p = jnp.exp(logits - m)
    g_ref[...] = p / jnp.sum(p, axis=-1, keepdims=True)


def _moe_body(x_ref, g_ref, w_ref, b_ref, out_ref, *, n_experts):
    xb = x_ref[...].astype(jnp.bfloat16)
    g = g_ref[...]
    acc = jnp.dot(g, b_ref[...], preferred_element_type=jnp.float32)
    for e in range(n_experts):
        ye = jnp.dot(xb, w_ref[e].astype(jnp.bfloat16), preferred_element_type=jnp.float32)
        acc = acc + g[:, e : e + 1] * ye
    out_ref[...] = acc


def kernel(x, gate_W, gate_b, expert_W, expert_b):
    n_tok, k = x.shape
    n_exp, _, n_out = expert_W.shape

    bm = min(1024, n_tok)
    bn = min(256, n_out)
    gb2 = gate_b.reshape(1, n_exp)

    g = pl.pallas_call(
        _gate_body,
        grid=(n_tok // bm,),
        in_specs=[
            pl.BlockSpec((bm, k), lambda m: (m, 0)),
            pl.BlockSpec((k, n_exp), lambda m: (0, 0)),
            pl.BlockSpec((1, n_exp), lambda m: (0, 0)),
        ],
        out_specs=pl.BlockSpec((bm, n_exp), lambda m: (m, 0)),
        out_shape=jax.ShapeDtypeStruct((n_tok, n_exp), jnp.float32),
    )(x, gate_W, gb2)

    body = functools.partial(_moe_body, n_experts=n_exp)
    return pl.pallas_call(
        body,
        grid=(n_out // bn, n_tok // bm),
        in_specs=[
            pl.BlockSpec((bm, k), lambda n, m: (m, 0)),
            pl.BlockSpec((bm, n_exp), lambda n, m: (m, 0)),
            pl.BlockSpec((n_exp, k, bn), lambda n, m: (0, 0, n)),
            pl.BlockSpec((n_exp, bn), lambda n, m: (0, n)),
        ],
        out_specs=pl.BlockSpec((bm, bn), lambda n, m: (m, n)),
        out_shape=jax.ShapeDtypeStruct((n_tok, n_out), jnp.float32),
        compiler_params=pltpu.CompilerParams(
            dimension_semantics=("arbitrary", "arbitrary"),
        ),
    )(x, g, expert_W, expert_b)


# speedup vs baseline: 1.0262x; 1.0119x over previous
import functools

import jax
import jax.numpy as jnp
from jax.experimental import pallas as pl
from jax.experimental.pallas import tpu as pltpu


def _moe_gate_body(x_ref, gw_ref, gb_ref, w_ref, b_ref, out_ref, g_ref, *, n_experts):
    xb = x_ref[...].astype(jnp.bfloat16)
    logits = jax.lax.dot_general(
        xb,
        gw_ref[...].astype(jnp.bfloat16),
        (((1,), (1,)), ((), ())),
        preferred_element_type=jnp.float32,
    )
    logits = logits + gb_ref[...]
    mx = jnp.max(logits, axis=-1, keepdims=True)
    p = jnp.exp(logits - mx)
    g = p / jnp.sum(p, axis=-1, keepdims=True)
    g_ref[...] = g

    acc = jnp.dot(g, b_ref[...], preferred_element_type=jnp.float32)
    for e in range(n_experts):
        ye = jnp.dot(xb, w_ref[e].astype(jnp.bfloat16), preferred_element_type=jnp.float32)
        acc = acc + g[:, e : e + 1] * ye
    out_ref[...] = acc


def _moe_body(x_ref, g_ref, w_ref, b_ref, out_init_ref, out_ref, *, n_experts):
    del out_init_ref
    xb = x_ref[...].astype(jnp.bfloat16)
    g = g_ref[...]
    acc = jnp.dot(g, b_ref[...], preferred_element_type=jnp.float32)
    for e in range(n_experts):
        ye = jnp.dot(xb, w_ref[e].astype(jnp.bfloat16), preferred_element_type=jnp.float32)
        acc = acc + g[:, e : e + 1] * ye
    out_ref[...] = acc


def kernel(x, gate_W, gate_b, expert_W, expert_b):
    n_tok, k = x.shape
    n_exp, _, n_out = expert_W.shape

    bm = min(1024, n_tok)
    bn = min(256, n_out)
    gb2 = gate_b.reshape(1, n_exp)
    gw_t = gate_W.T

    gate_body = functools.partial(_moe_gate_body, n_experts=n_exp)
    out0, g = pl.pallas_call(
        gate_body,
        grid=(n_tok // bm,),
        in_specs=[
            pl.BlockSpec((bm, k), lambda m: (m, 0)),
            pl.BlockSpec((n_exp, k), lambda m: (0, 0)),
            pl.BlockSpec((1, n_exp), lambda m: (0, 0)),
            pl.BlockSpec((n_exp, k, bn), lambda m: (0, 0, 0)),
            pl.BlockSpec((n_exp, bn), lambda m: (0, 0)),
        ],
        out_specs=(
            pl.BlockSpec((bm, bn), lambda m: (m, 0)),
            pl.BlockSpec((bm, n_exp), lambda m: (m, 0)),
        ),
        out_shape=(
            jax.ShapeDtypeStruct((n_tok, n_out), jnp.float32),
            jax.ShapeDtypeStruct((n_tok, n_exp), jnp.float32),
        ),
        compiler_params=pltpu.CompilerParams(
            dimension_semantics=("arbitrary",),
        ),
    )(x, gw_t, gb2, expert_W, expert_b)

    body = functools.partial(_moe_body, n_experts=n_exp)
    return pl.pallas_call(
        body,
        grid=(n_out // bn - 1, n_tok // bm),
        in_specs=[
            pl.BlockSpec((bm, k), lambda n, m: (m, 0)),
            pl.BlockSpec((bm, n_exp), lambda n, m: (m, 0)),
            pl.BlockSpec((n_exp, k, bn), lambda n, m: (0, 0, n + 1)),
            pl.BlockSpec((n_exp, bn), lambda n, m: (0, n + 1)),
            pl.BlockSpec(memory_space=pltpu.MemorySpace.HBM),
        ],
        out_specs=pl.BlockSpec((bm, bn), lambda n, m: (m, n + 1)),
        out_shape=jax.ShapeDtypeStruct((n_tok, n_out), jnp.float32),
        input_output_aliases={4: 0},
        compiler_params=pltpu.CompilerParams(
            dimension_semantics=("arbitrary", "arbitrary"),
        ),
    )(x, g, expert_W, expert_b, out0)
